# baseline (speedup 1.0000x reference)
import jax
import jax.numpy as jnp
from jax.experimental import pallas as pl
from jax.experimental.pallas import tpu as pltpu

_B, _L, _DIN = 8, 2048, 16
_NT, _PD = 6, 6
_DM = 40
_T = _DIN * _L


def _f32(x):
    return x.astype(jnp.float32)


def _body(inp_hbm, dates_hbm, cmax_hbm, tw_hbm, tb_hbm, local_hbm, space_hbm,
          out_ref, var_ref,
          inp_s, dates_s, cmax_s, tw_s, tb_s, local_s, space_s, base_ref, sem):
    b = pl.program_id(0)

    @pl.when(b == 0)
    def _():
        copies = [
            pltpu.make_async_copy(inp_hbm, inp_s, sem),
            pltpu.make_async_copy(dates_hbm, dates_s, sem),
            pltpu.make_async_copy(cmax_hbm, cmax_s, sem),
            pltpu.make_async_copy(tw_hbm, tw_s, sem),
            pltpu.make_async_copy(tb_hbm, tb_s, sem),
            pltpu.make_async_copy(local_hbm.at[:, : _L], local_s, sem),
            pltpu.make_async_copy(space_hbm, space_s, sem),
        ]
        for c in copies:
            c.start()
        for c in copies:
            c.wait()
        tvar = jax.lax.broadcasted_iota(jnp.int32, (_B, _T), 1)
        var_ref[...] = jax.lax.shift_right_logical(tvar, 11)

    c40_6 = jax.lax.broadcasted_iota(jnp.int32, (_DM, _NT), 0)
    i40_6 = jax.lax.broadcasted_iota(jnp.int32, (_DM, _NT), 1)
    trow = (c40_6 >= 1) & (c40_6 <= 36)
    u = _f32(trow & (i40_6 == (c40_6 - 1) // _PD))
    j6_40 = jax.lax.broadcasted_iota(jnp.int32, (_NT, _DM), 0)
    c6_40 = jax.lax.broadcasted_iota(jnp.int32, (_NT, _DM), 1)
    v = _f32((c6_40 >= 1) & (c6_40 <= 36) & (j6_40 == (c6_40 - 1) % _PD))
    c40_3 = jax.lax.broadcasted_iota(jnp.int32, (_DM, 3), 0)
    h40_3 = jax.lax.broadcasted_iota(jnp.int32, (_DM, 3), 1)
    s3t = _f32(c40_3 == 37 + h40_3)

    dates_b = dates_s[:, b, :]
    cmax_b = cmax_s[:, b, :]

    dw = jnp.dot(jnp.dot(u, tw_s[...], preferred_element_type=jnp.float32),
                 v, preferred_element_type=jnp.float32)
    db = jnp.dot(jnp.dot(u, tb_s[...], preferred_element_type=jnp.float32),
                 v, preferred_element_type=jnp.float32)
    r0 = jax.lax.broadcasted_iota(jnp.int32, (_DM, _DM), 0)
    r1 = jax.lax.broadcasted_iota(jnp.int32, (_DM, _DM), 1)
    eye = _f32(r0 == r1)
    w40 = jnp.sum(dw * eye, axis=1, keepdims=True)
    b40 = jnp.sum(db * eye, axis=1, keepdims=True)

    dates_spread = jnp.dot(u, dates_b, preferred_element_type=jnp.float32)
    lin = (dates_spread * w40 + b40
           + jnp.dot(s3t, cmax_b, preferred_element_type=jnp.float32))
    row = jax.lax.broadcasted_iota(jnp.int32, (_DM, 1), 0)
    sinmask = (row >= 1) & (row <= 36) & ((row - 1) % _PD != 0)
    base_ref[...] = local_s[...] + jnp.where(sinmask, jnp.sin(lin), lin)

    base0 = base_ref[0:1, :]
    inp_b = inp_s[b]
    for d in range(_DIN):
        sl = pl.ds(d * _L, _L)
        out_ref[0, :, sl] = base_ref[...]
        out_ref[0, 0:1, sl] = (base0 + inp_b[d:d + 1, :]
                               + space_s[0:1, d:d + 1])


def kernel(input, dates, cmax, time_w, time_b, local_emb, space_emb):
    inp_t = jnp.transpose(input, (0, 2, 1))
    dates_t = jnp.transpose(dates, (2, 0, 1))
    cmax_t = jnp.transpose(cmax, (2, 0, 1))
    local_t = jnp.transpose(local_emb)
    space_t = jnp.transpose(space_emb)

    any_spec = pl.BlockSpec(memory_space=pl.ANY)
    out_cm, var = pl.pallas_call(
        _body,
        grid=(_B,),
        in_specs=[any_spec] * 7,
        out_specs=[
            pl.BlockSpec((1, _DM, _T), lambda b: (b, 0, 0)),
            pl.BlockSpec((_B, _T), lambda b: (0, 0)),
        ],
        out_shape=[
            jax.ShapeDtypeStruct((_B, _DM, _T), jnp.float32),
            jax.ShapeDtypeStruct((_B, _T), jnp.int32),
        ],
        scratch_shapes=[
            pltpu.VMEM((_B, _DIN, _L), jnp.float32),
            pltpu.VMEM((_NT, _B, _L), jnp.float32),
            pltpu.VMEM((3, _B, _L), jnp.float32),
            pltpu.VMEM((_NT, _NT), jnp.float32),
            pltpu.VMEM((_NT, _NT), jnp.float32),
            pltpu.VMEM((_DM, _L), jnp.float32),
            pltpu.VMEM((1, _DIN), jnp.float32),
            pltpu.VMEM((_DM, _L), jnp.float32),
            pltpu.SemaphoreType.DMA,
        ],
        compiler_params=pltpu.CompilerParams(
            dimension_semantics=("arbitrary",),
            vmem_limit_bytes=50 * 1024 * 1024,
        ),
    )(inp_t, dates_t, cmax_t, time_w, time_b, local_t, space_t)
    out = jnp.transpose(out_cm, (0, 2, 1))
    return out, var


# speedup vs baseline: 1.1673x; 1.1673x over previous
import numpy as np
import jax
import jax.numpy as jnp
from jax.experimental import pallas as pl
from jax.experimental.pallas import tpu as pltpu

_B, _L, _DIN = 8, 2048, 16
_NT, _PD = 6, 6
_DM = 40
_T = _DIN * _L

_TT = np.linspace(-np.pi, np.pi, 20001)
_A = np.stack([_TT, _TT**3, _TT**5, _TT**7, _TT**9, _TT**11], axis=1)
_C = np.linalg.lstsq(_A, np.sin(_TT), rcond=None)[0].astype(np.float32)
_INV2PI = np.float32(1.0 / (2.0 * np.pi))
_TWOPI = np.float32(2.0 * np.pi)


def _psin(x):
    y = x * _INV2PI
    t = (y - jnp.round(y)) * _TWOPI
    t2 = t * t
    p = _C[5]
    for c in (_C[4], _C[3], _C[2], _C[1], _C[0]):
        p = p * t2 + c
    return t * p


def _f32(x):
    return x.astype(jnp.float32)


def _body(inp_ref, dates_ref, cmax_ref, tw_ref, tb_ref, local_ref, space_ref,
          out_ref, var_ref, base_ref):
    b = pl.program_id(0)

    @pl.when(b == 0)
    def _():
        tvar = jax.lax.broadcasted_iota(jnp.int32, (_B, _T), 1)
        var_ref[...] = jax.lax.shift_right_logical(tvar, 11)

    c40_6 = jax.lax.broadcasted_iota(jnp.int32, (_DM, _NT), 0)
    i40_6 = jax.lax.broadcasted_iota(jnp.int32, (_DM, _NT), 1)
    trow = (c40_6 >= 1) & (c40_6 <= 36)
    u = _f32(trow & (i40_6 == (c40_6 - 1) // _PD))
    j6_40 = jax.lax.broadcasted_iota(jnp.int32, (_NT, _DM), 0)
    c6_40 = jax.lax.broadcasted_iota(jnp.int32, (_NT, _DM), 1)
    v = _f32((c6_40 >= 1) & (c6_40 <= 36) & (j6_40 == (c6_40 - 1) % _PD))
    c40_3 = jax.lax.broadcasted_iota(jnp.int32, (_DM, 3), 0)
    h40_3 = jax.lax.broadcasted_iota(jnp.int32, (_DM, 3), 1)
    s3t = _f32(c40_3 == 37 + h40_3)

    bsub6 = jax.lax.broadcasted_iota(jnp.int32, (_NT, _B, _L), 1)
    dates_b = jnp.sum(jnp.where(bsub6 == b, dates_ref[...], 0.0), axis=1)
    bsub3 = jax.lax.broadcasted_iota(jnp.int32, (3, _B, _L), 1)
    cmax_b = jnp.sum(jnp.where(bsub3 == b, cmax_ref[...], 0.0), axis=1)

    dw = jnp.dot(jnp.dot(u, tw_ref[...], preferred_element_type=jnp.float32),
                 v, preferred_element_type=jnp.float32)
    db = jnp.dot(jnp.dot(u, tb_ref[...], preferred_element_type=jnp.float32),
                 v, preferred_element_type=jnp.float32)
    r0 = jax.lax.broadcasted_iota(jnp.int32, (_DM, _DM), 0)
    r1 = jax.lax.broadcasted_iota(jnp.int32, (_DM, _DM), 1)
    eye = _f32(r0 == r1)
    w40 = jnp.sum(dw * eye, axis=1, keepdims=True)
    b40 = jnp.sum(db * eye, axis=1, keepdims=True)

    dates_spread = jnp.dot(u, dates_b, preferred_element_type=jnp.float32)
    lin = (dates_spread * w40 + b40
           + jnp.dot(s3t, cmax_b, preferred_element_type=jnp.float32))
    row = jax.lax.broadcasted_iota(jnp.int32, (_DM, 1), 0)
    sinmask = (row >= 1) & (row <= 36) & ((row - 1) % _PD != 0)
    base_ref[...] = local_ref[...] + jnp.where(sinmask, _psin(lin), lin)

    base0 = base_ref[0:1, :]
    inp_b = inp_ref[0]
    for d in range(_DIN):
        sl = pl.ds(d * _L, _L)
        out_ref[0, :, sl] = base_ref[...]
        out_ref[0, 0:1, sl] = (base0 + inp_b[d:d + 1, :]
                               + space_ref[0:1, d:d + 1])


def kernel(input, dates, cmax, time_w, time_b, local_emb, space_emb):
    inp_t = jnp.transpose(input, (0, 2, 1))
    dates_t = jnp.transpose(dates, (2, 0, 1))
    cmax_t = jnp.transpose(cmax, (2, 0, 1))
    local_t = jnp.transpose(local_emb)
    space_t = jnp.transpose(space_emb)

    out_cm, var = pl.pallas_call(
        _body,
        grid=(_B,),
        in_specs=[
            pl.BlockSpec((1, _DIN, _L), lambda b: (b, 0, 0)),
            pl.BlockSpec((_NT, _B, _L), lambda b: (0, 0, 0)),
            pl.BlockSpec((3, _B, _L), lambda b: (0, 0, 0)),
            pl.BlockSpec((_NT, _NT), lambda b: (0, 0)),
            pl.BlockSpec((_NT, _NT), lambda b: (0, 0)),
            pl.BlockSpec((_DM, _L), lambda b: (0, 0)),
            pl.BlockSpec((1, _DIN), lambda b: (0, 0)),
        ],
        out_specs=[
            pl.BlockSpec((1, _DM, _T), lambda b: (b, 0, 0)),
            pl.BlockSpec((_B, _T), lambda b: (0, 0)),
        ],
        out_shape=[
            jax.ShapeDtypeStruct((_B, _DM, _T), jnp.float32),
            jax.ShapeDtypeStruct((_B, _T), jnp.int32),
        ],
        scratch_shapes=[pltpu.VMEM((_DM, _L), jnp.float32)],
        compiler_params=pltpu.CompilerParams(
            dimension_semantics=("arbitrary",),
            vmem_limit_bytes=50 * 1024 * 1024,
        ),
    )(inp_t, dates_t, cmax_t, time_w, time_b, local_t, space_t)
    out = jnp.transpose(out_cm, (0, 2, 1))
    return out, var
